# baseline (speedup 1.0000x reference)
import jax
import jax.numpy as jnp
from jax.experimental import pallas as pl
from jax.experimental.pallas import tpu as pltpu

_K = 1024


def _body(x_ref, lens_ref, w1_ref, b1_ref, w2_ref,
          act_ref, lp_ref, en_ref, sc_ref, carry_ref):
    c = pl.program_id(0)
    nc = pl.num_programs(0)
    B, K, D = x_ref.shape
    L = B * nc

    @pl.when(c == 0)
    def _init():
        carry_ref[...] = jnp.zeros_like(carry_ref)

    x = x_ref[...]
    x_shift = jnp.concatenate([carry_ref[...], x[:, :-1, :]], axis=1)
    pairs = jnp.concatenate([x_shift, x], axis=2).astype(jnp.bfloat16)
    w1b = w1_ref[...].astype(jnp.bfloat16)
    r = jax.lax.dot_general(pairs.reshape(B * K, 2 * D), w1b,
                            (((1,), (1,)), ((), ())),
                            preferred_element_type=jnp.float32)
    h = jnp.maximum(r + b1_ref[...], 0.0).astype(jnp.bfloat16)
    h3 = h.reshape(B, K, w1b.shape[0])
    w2rep = jnp.broadcast_to(jnp.transpose(w2_ref[...].astype(jnp.bfloat16)),
                             (w1b.shape[0], L))
    lane = jax.lax.broadcasted_iota(jnp.int32, (K, L), 1)
    acc = sc_ref[...]
    for b in range(B):
        vb = jnp.dot(h3[b], w2rep, preferred_element_type=jnp.float32)
        acc = jnp.where(lane == c * B + b, vb, acc)
    sc_ref[...] = acc
    carry_ref[...] = x[:, -1:, :]

    @pl.when(c == nc - 1)
    def _finalize():
        s2 = sc_ref[...]
        q = jax.lax.broadcasted_iota(jnp.int32, (K, L), 0)
        j = jax.lax.broadcasted_iota(jnp.int32, (K, L), 1)
        pos = (j // B) * K + q - 1
        lens_l = jnp.tile(lens_ref[...], (1, nc))
        valid = (pos >= 0) & (pos < lens_l - 1)
        s_m = jnp.where(valid, s2, -jnp.inf)

        def groupred(vec, op):
            sh = B
            while sh < L:
                vec = op(vec, pltpu.roll(vec, sh, 1))
                sh *= 2
            return vec

        m = groupred(jnp.max(s_m, axis=0, keepdims=True), jnp.maximum)
        e = jnp.exp(s_m - m)
        l = groupred(jnp.sum(e, axis=0, keepdims=True), jnp.add)
        t = groupred(jnp.sum(e * s2, axis=0, keepdims=True), jnp.add)
        cand = jnp.where(s_m == m, pos, jnp.int32(2**30))
        pmin = groupred(jnp.min(cand, axis=0, keepdims=True), jnp.minimum)
        le = lens_ref[...]
        empty = le <= 1
        nan = jnp.float32(jnp.nan)
        logl = jnp.log(l[:, :B])
        act_ref[...] = jnp.where(empty, 0, pmin[:, :B])
        lp_ref[...] = jnp.where(empty, nan, -logl)
        en_ref[...] = jnp.where(empty, nan, m[:, :B] + logl - t[:, :B] / l[:, :B])


def kernel(sequence_embedding, sentence_lengths, W1, b1, W2, b2):
    B, S, D = sequence_embedding.shape
    H = W1.shape[0]
    K = _K
    NC = S // K

    act, lp, en = pl.pallas_call(
        _body,
        grid=(NC,),
        in_specs=[
            pl.BlockSpec((B, K, D), lambda c: (0, c, 0)),
            pl.BlockSpec((1, B), lambda c: (0, 0)),
            pl.BlockSpec((H, 2 * D), lambda c: (0, 0)),
            pl.BlockSpec((1, H), lambda c: (0, 0)),
            pl.BlockSpec((1, H), lambda c: (0, 0)),
        ],
        out_specs=[
            pl.BlockSpec((1, B), lambda c: (0, 0)),
            pl.BlockSpec((1, B), lambda c: (0, 0)),
            pl.BlockSpec((1, B), lambda c: (0, 0)),
        ],
        out_shape=[
            jax.ShapeDtypeStruct((1, B), jnp.int32),
            jax.ShapeDtypeStruct((1, B), jnp.float32),
            jax.ShapeDtypeStruct((1, B), jnp.float32),
        ],
        scratch_shapes=[
            pltpu.VMEM((K, B * NC), jnp.float32),
            pltpu.VMEM((B, 1, D), jnp.float32),
        ],
    )(sequence_embedding, sentence_lengths.reshape(1, B), W1, b1.reshape(1, H),
      W2)
    return act[0], lp[0], en[0]


# speedup vs baseline: 1.0180x; 1.0180x over previous
import jax
import jax.numpy as jnp
from jax.experimental import pallas as pl
from jax.experimental.pallas import tpu as pltpu

_K = 512


def _body(x_ref, lens_ref, w1_ref, b1_ref, w2_ref,
          act_ref, lp_ref, en_ref, sc_ref, carry_ref):
    c = pl.program_id(0)
    nc = pl.num_programs(0)
    B, K, D = x_ref.shape
    L = B * nc

    @pl.when(c == 0)
    def _init():
        carry_ref[...] = jnp.zeros_like(carry_ref)

    x = x_ref[...]
    x_shift = jnp.concatenate([carry_ref[...], x[:, :-1, :]], axis=1)
    pairs = jnp.concatenate([x_shift, x], axis=2).astype(jnp.bfloat16)
    w1b = w1_ref[...].astype(jnp.bfloat16)
    r = jax.lax.dot_general(pairs.reshape(B * K, 2 * D), w1b,
                            (((1,), (1,)), ((), ())),
                            preferred_element_type=jnp.float32)
    h = jnp.maximum(r + b1_ref[...], 0.0).astype(jnp.bfloat16)
    h3 = h.reshape(B, K, w1b.shape[0])
    w2rep = jnp.broadcast_to(jnp.transpose(w2_ref[...].astype(jnp.bfloat16)),
                             (w1b.shape[0], L))
    lane = jax.lax.broadcasted_iota(jnp.int32, (K, L), 1)
    acc = sc_ref[...]
    for b in range(B):
        vb = jnp.dot(h3[b], w2rep, preferred_element_type=jnp.float32)
        acc = jnp.where(lane == c * B + b, vb, acc)
    sc_ref[...] = acc
    carry_ref[...] = x[:, -1:, :]

    @pl.when(c == nc - 1)
    def _finalize():
        s2 = sc_ref[...]
        q = jax.lax.broadcasted_iota(jnp.int32, (K, L), 0)
        j = jax.lax.broadcasted_iota(jnp.int32, (K, L), 1)
        pos = (j // B) * K + q - 1
        lens_l = jnp.tile(lens_ref[...], (1, nc))
        valid = (pos >= 0) & (pos < lens_l - 1)
        s_m = jnp.where(valid, s2, -jnp.inf)

        def groupred(vec, op):
            sh = B
            while sh < L:
                vec = op(vec, pltpu.roll(vec, sh, 1))
                sh *= 2
            return vec

        m = groupred(jnp.max(s_m, axis=0, keepdims=True), jnp.maximum)
        e = jnp.exp(s_m - m)
        l = groupred(jnp.sum(e, axis=0, keepdims=True), jnp.add)
        t = groupred(jnp.sum(e * s2, axis=0, keepdims=True), jnp.add)
        cand = jnp.where(s_m == m, pos, jnp.int32(2**30))
        pmin = groupred(jnp.min(cand, axis=0, keepdims=True), jnp.minimum)
        le = lens_ref[...]
        empty = le <= 1
        nan = jnp.float32(jnp.nan)
        logl = jnp.log(l[:, :B])
        act_ref[...] = jnp.where(empty, 0, pmin[:, :B])
        lp_ref[...] = jnp.where(empty, nan, -logl)
        en_ref[...] = jnp.where(empty, nan, m[:, :B] + logl - t[:, :B] / l[:, :B])


def kernel(sequence_embedding, sentence_lengths, W1, b1, W2, b2):
    B, S, D = sequence_embedding.shape
    H = W1.shape[0]
    K = _K
    NC = S // K

    act, lp, en = pl.pallas_call(
        _body,
        grid=(NC,),
        in_specs=[
            pl.BlockSpec((B, K, D), lambda c: (0, c, 0)),
            pl.BlockSpec((1, B), lambda c: (0, 0)),
            pl.BlockSpec((H, 2 * D), lambda c: (0, 0)),
            pl.BlockSpec((1, H), lambda c: (0, 0)),
            pl.BlockSpec((1, H), lambda c: (0, 0)),
        ],
        out_specs=[
            pl.BlockSpec((1, B), lambda c: (0, 0)),
            pl.BlockSpec((1, B), lambda c: (0, 0)),
            pl.BlockSpec((1, B), lambda c: (0, 0)),
        ],
        out_shape=[
            jax.ShapeDtypeStruct((1, B), jnp.int32),
            jax.ShapeDtypeStruct((1, B), jnp.float32),
            jax.ShapeDtypeStruct((1, B), jnp.float32),
        ],
        scratch_shapes=[
            pltpu.VMEM((K, B * NC), jnp.float32),
            pltpu.VMEM((B, 1, D), jnp.float32),
        ],
    )(sequence_embedding, sentence_lengths.reshape(1, B), W1, b1.reshape(1, H),
      W2)
    return act[0], lp[0], en[0]
